# Initial kernel scaffold; baseline (speedup 1.0000x reference)
import jax
import jax.numpy as jnp
from jax.experimental import pallas as pl


def kernel(x_nchw, w1, g1, b1, w2, g2, b2, w3, b3c, g3, b3, wd, bd, gd, betad):
    raise NotImplementedError("write your pallas kernel here")


# speedup vs baseline: 3.2423x; 3.2423x over previous
import jax
import jax.numpy as jnp
from jax.experimental import pallas as pl
from jax.experimental.pallas import tpu as pltpu

_EPS = 1e-5
_VMEM_LIMIT = 48 * 1024 * 1024


def _bn_fold(s, q, gamma, beta, m_real):
    mean = s / float(m_real)
    var = jnp.maximum(q / float(m_real) - mean * mean, 0.0)
    rstd = jax.lax.rsqrt(var + _EPS)
    scale = gamma.reshape(1, -1) * rstd
    shift = beta.reshape(1, -1) - mean * scale
    return jnp.concatenate([scale, shift], axis=0)


def _k_stats1(x_ref, w_ref, s_ref, q_ref):
    y = jnp.dot(x_ref[...], w_ref[...], preferred_element_type=jnp.float32)
    s_ref[...] = jnp.sum(y, axis=0, keepdims=True)[None]
    q_ref[...] = jnp.sum(y * y, axis=0, keepdims=True)[None]


def _k_stats2(y_ref, a_ref, w_ref, s_ref, q_ref):
    h = jnp.maximum(y_ref[...] * a_ref[0:1, :] + a_ref[1:2, :], 0.0)
    y3 = jnp.dot(h, w_ref[...], preferred_element_type=jnp.float32)
    s_ref[...] = jnp.sum(y3, axis=0, keepdims=True)[None]
    q_ref[...] = jnp.sum(y3 * y3, axis=0, keepdims=True)[None]


def _make_conv2_kernel(H, W, C, HP_ROWS, BASE):
    HW = H * W

    def _k_conv2(x_ref, a_ref, w1_ref, w2_ref, y2_ref, s_ref, q_ref,
                 hp_ref, pat_ref):
        y1 = jnp.dot(x_ref[...], w1_ref[...],
                     preferred_element_type=jnp.float32)
        h = jnp.maximum(y1 * a_ref[0:1, :] + a_ref[1:2, :], 0.0)
        hp_ref[0:BASE, :] = jnp.zeros((BASE, C), jnp.float32)
        hp_ref[BASE + HW:HP_ROWS, :] = jnp.zeros(
            (HP_ROWS - BASE - HW, C), jnp.float32)
        hp_ref[BASE:BASE + HW, :] = h
        col = jax.lax.broadcasted_iota(jnp.int32, (HW, 1), 0) % W
        m0 = (col != 0).astype(jnp.float32)
        m2 = (col != W - 1).astype(jnp.float32)
        for di in range(3):
            for dj in range(3):
                t = 3 * di + dj
                off = W * di + dj
                win = hp_ref[off:off + HW, :]
                if dj == 0:
                    win = win * m0
                elif dj == 2:
                    win = win * m2
                pat_ref[:, t * C:(t + 1) * C] = win
        y2 = jnp.dot(pat_ref[...], w2_ref[...],
                     preferred_element_type=jnp.float32)
        y2_ref[...] = y2
        s_ref[...] = jnp.sum(y2, axis=0, keepdims=True)[None]
        q_ref[...] = jnp.sum(y2 * y2, axis=0, keepdims=True)[None]

    return _k_conv2


def _k_out(y2_ref, a2_ref, w3_ref, a3_ref, x_ref, o_ref):
    h = jnp.maximum(y2_ref[...] * a2_ref[0:1, :] + a2_ref[1:2, :], 0.0)
    y3 = jnp.dot(h, w3_ref[...], preferred_element_type=jnp.float32)
    o_ref[...] = jnp.maximum(
        y3 * a3_ref[0:1, :] + a3_ref[1:2, :] + x_ref[...], 0.0)


def kernel(x_nchw, w1, g1, b1, w2, g2, b2, w3, b3c, g3, b3,
           wd, bd, gd, betad):
    N, C, H, W = x_nchw.shape
    HW = H * W
    M = N * HW
    f32 = jnp.float32

    x = jnp.transpose(x_nchw, (0, 2, 3, 1)).reshape(M, C).astype(f32)
    width = w1.shape[0]
    c_out = w3.shape[0]
    w1m = w1.reshape(width, C).T.astype(f32)
    w2m = jnp.transpose(w2.astype(f32), (2, 3, 1, 0)).reshape(9 * width, width)
    w3m = w3.reshape(c_out, width).T.astype(f32)

    for TM in (512, 448, 256):
        if M % TM == 0:
            break
    else:
        TM = HW
    nm = M // TM

    BASE = W + 1
    HP_ROWS = -(-(2 * W + 2 + HW + BASE) // 8) * 8

    stat_sds = jax.ShapeDtypeStruct
    params = pltpu.CompilerParams(
        dimension_semantics=("parallel",), vmem_limit_bytes=_VMEM_LIMIT)

    s1p, q1p = pl.pallas_call(
        _k_stats1,
        out_shape=(stat_sds((nm, 1, width), f32), stat_sds((nm, 1, width), f32)),
        grid=(nm,),
        in_specs=[pl.BlockSpec((TM, C), lambda m: (m, 0)),
                  pl.BlockSpec((C, width), lambda m: (0, 0))],
        out_specs=(pl.BlockSpec((1, 1, width), lambda m: (m, 0, 0)),
                   pl.BlockSpec((1, 1, width), lambda m: (m, 0, 0))),
        compiler_params=params,
    )(x, w1m)
    a1 = _bn_fold(jnp.sum(s1p, axis=0), jnp.sum(q1p, axis=0), g1, b1, M)

    y2, s2p, q2p = pl.pallas_call(
        _make_conv2_kernel(H, W, C, HP_ROWS, BASE),
        out_shape=(stat_sds((M, width), f32),
                   stat_sds((N, 1, width), f32),
                   stat_sds((N, 1, width), f32)),
        grid=(N,),
        in_specs=[pl.BlockSpec((HW, C), lambda n: (n, 0)),
                  pl.BlockSpec((2, width), lambda n: (0, 0)),
                  pl.BlockSpec((C, width), lambda n: (0, 0)),
                  pl.BlockSpec((9 * width, width), lambda n: (0, 0))],
        out_specs=(pl.BlockSpec((HW, width), lambda n: (n, 0)),
                   pl.BlockSpec((1, 1, width), lambda n: (n, 0, 0)),
                   pl.BlockSpec((1, 1, width), lambda n: (n, 0, 0))),
        scratch_shapes=[pltpu.VMEM((HP_ROWS, C), f32),
                        pltpu.VMEM((HW, 9 * width), f32)],
        compiler_params=params,
    )(x, a1, w1m, w2m)
    a2 = _bn_fold(jnp.sum(s2p, axis=0), jnp.sum(q2p, axis=0), g2, b2, M)

    s3p, q3p = pl.pallas_call(
        _k_stats2,
        out_shape=(stat_sds((nm, 1, c_out), f32), stat_sds((nm, 1, c_out), f32)),
        grid=(nm,),
        in_specs=[pl.BlockSpec((TM, width), lambda m: (m, 0)),
                  pl.BlockSpec((2, width), lambda m: (0, 0)),
                  pl.BlockSpec((width, c_out), lambda m: (0, 0))],
        out_specs=(pl.BlockSpec((1, 1, c_out), lambda m: (m, 0, 0)),
                   pl.BlockSpec((1, 1, c_out), lambda m: (m, 0, 0))),
        compiler_params=params,
    )(y2, a2, w3m)
    a3 = _bn_fold(jnp.sum(s3p, axis=0), jnp.sum(q3p, axis=0), g3, b3, M)

    out = pl.pallas_call(
        _k_out,
        out_shape=stat_sds((M, c_out), f32),
        grid=(nm,),
        in_specs=[pl.BlockSpec((TM, width), lambda m: (m, 0)),
                  pl.BlockSpec((2, width), lambda m: (0, 0)),
                  pl.BlockSpec((width, c_out), lambda m: (0, 0)),
                  pl.BlockSpec((2, c_out), lambda m: (0, 0)),
                  pl.BlockSpec((TM, C), lambda m: (m, 0))],
        out_specs=pl.BlockSpec((TM, c_out), lambda m: (m, 0)),
        compiler_params=params,
    )(y2, a2, w3m, a3, x)

    return jnp.transpose(out.reshape(N, H, W, c_out), (0, 3, 1, 2))


# speedup vs baseline: 4.5390x; 1.3999x over previous
import jax
import jax.numpy as jnp
from jax.experimental import pallas as pl
from jax.experimental.pallas import tpu as pltpu

_EPS = 1e-5
_VMEM_LIMIT = 48 * 1024 * 1024


def _bn_fold(s, q, gamma, beta, m_real):
    mean = s / float(m_real)
    var = jnp.maximum(q / float(m_real) - mean * mean, 0.0)
    rstd = jax.lax.rsqrt(var + _EPS)
    scale = gamma.reshape(1, -1) * rstd
    shift = beta.reshape(1, -1) - mean * scale
    return jnp.concatenate([scale, shift], axis=0)


def _k_gram(x_ref, g_ref, sx_ref):
    b = x_ref.shape[0]
    acc = jnp.zeros((x_ref.shape[1], x_ref.shape[1]), jnp.float32)
    sx = jnp.zeros((x_ref.shape[1], 1), jnp.float32)
    for i in range(b):
        xi = x_ref[i]
        acc = acc + jax.lax.dot_general(
            xi, xi, (((1,), (1,)), ((), ())),
            preferred_element_type=jnp.float32)
        sx = sx + jnp.sum(xi, axis=1, keepdims=True)
    g_ref[...] = acc[None]
    sx_ref[...] = sx[None]


def _k_stats2(y_ref, a_ref, w_ref, s_ref, q_ref):
    h = jnp.maximum(y_ref[...] * a_ref[0:1, :] + a_ref[1:2, :], 0.0)
    y3 = jnp.dot(h, w_ref[...], preferred_element_type=jnp.float32)
    s_ref[...] = jnp.sum(y3, axis=0, keepdims=True)[None]
    q_ref[...] = jnp.sum(y3 * y3, axis=0, keepdims=True)[None]


def _make_conv2_kernel(H, W, C, HP_ROWS, BASE):
    HW = H * W

    def _k_conv2(x_ref, a_ref, w1_ref, w2_ref, y2_ref, s_ref, q_ref,
                 hp_ref, pat_ref):
        y1 = jax.lax.dot_general(
            x_ref[0], w1_ref[...], (((0,), (0,)), ((), ())),
            preferred_element_type=jnp.float32)
        h = jnp.maximum(y1 * a_ref[0:1, :] + a_ref[1:2, :], 0.0)
        hp_ref[0:BASE, :] = jnp.zeros((BASE, h.shape[1]), jnp.float32)
        hp_ref[BASE + HW:HP_ROWS, :] = jnp.zeros(
            (HP_ROWS - BASE - HW, h.shape[1]), jnp.float32)
        hp_ref[BASE:BASE + HW, :] = h
        col = jax.lax.broadcasted_iota(jnp.int32, (HW, 1), 0) % W
        m0 = (col != 0).astype(jnp.float32)
        m2 = (col != W - 1).astype(jnp.float32)
        CW = h.shape[1]
        for di in range(3):
            for dj in range(3):
                t = 3 * di + dj
                off = W * di + dj
                win = hp_ref[off:off + HW, :]
                if dj == 0:
                    win = win * m0
                elif dj == 2:
                    win = win * m2
                pat_ref[:, t * CW:(t + 1) * CW] = win
        y2 = jnp.dot(pat_ref[...], w2_ref[...],
                     preferred_element_type=jnp.float32)
        y2_ref[...] = y2
        s_ref[...] = jnp.sum(y2, axis=0, keepdims=True)[None]
        q_ref[...] = jnp.sum(y2 * y2, axis=0, keepdims=True)[None]

    return _k_conv2


def _k_out(y2_ref, a2_ref, w3_ref, a3_ref, x_ref, o_ref):
    h = jnp.maximum(y2_ref[...] * a2_ref[0:1, :] + a2_ref[1:2, :], 0.0)
    y3 = jnp.dot(h, w3_ref[...], preferred_element_type=jnp.float32)
    res = jnp.transpose(x_ref[0], (1, 0))
    o = jnp.maximum(y3 * a3_ref[0:1, :] + a3_ref[1:2, :] + res, 0.0)
    o_ref[...] = jnp.transpose(o, (1, 0))[None]


def kernel(x_nchw, w1, g1, b1, w2, g2, b2, w3, b3c, g3, b3,
           wd, bd, gd, betad):
    N, C, H, W = x_nchw.shape
    HW = H * W
    M = N * HW
    f32 = jnp.float32

    xc = x_nchw.reshape(N, C, HW).astype(f32)
    width = w1.shape[0]
    c_out = w3.shape[0]
    w1m = w1.reshape(width, C).T.astype(f32)
    w2m = jnp.transpose(w2.astype(f32), (2, 3, 1, 0)).reshape(9 * width, width)
    w3m = w3.reshape(c_out, width).T.astype(f32)

    for TM in (512, 448, 256):
        if M % TM == 0:
            break
    else:
        TM = HW
    nm = M // TM

    GB = 4
    while N % GB:
        GB //= 2
    ng = N // GB

    BASE = W + 1
    HP_ROWS = -(-(2 * W + 2 + HW + BASE) // 8) * 8

    sds = jax.ShapeDtypeStruct
    params = pltpu.CompilerParams(
        dimension_semantics=("parallel",), vmem_limit_bytes=_VMEM_LIMIT)

    gp, sxp = pl.pallas_call(
        _k_gram,
        out_shape=(sds((ng, C, C), f32), sds((ng, C, 1), f32)),
        grid=(ng,),
        in_specs=[pl.BlockSpec((GB, C, HW), lambda i: (i, 0, 0))],
        out_specs=(pl.BlockSpec((1, C, C), lambda i: (i, 0, 0)),
                   pl.BlockSpec((1, C, 1), lambda i: (i, 0, 0))),
        compiler_params=params,
    )(xc)
    g1m = jnp.sum(gp, axis=0)
    sx = jnp.sum(sxp, axis=0).reshape(1, C)
    s1 = sx @ w1m
    q1 = jnp.sum(w1m * (g1m @ w1m), axis=0).reshape(1, width)
    a1 = _bn_fold(s1, q1, g1, b1, M)

    y2, s2p, q2p = pl.pallas_call(
        _make_conv2_kernel(H, W, C, HP_ROWS, BASE),
        out_shape=(sds((M, width), f32),
                   sds((N, 1, width), f32),
                   sds((N, 1, width), f32)),
        grid=(N,),
        in_specs=[pl.BlockSpec((1, C, HW), lambda n: (n, 0, 0)),
                  pl.BlockSpec((2, width), lambda n: (0, 0)),
                  pl.BlockSpec((C, width), lambda n: (0, 0)),
                  pl.BlockSpec((9 * width, width), lambda n: (0, 0))],
        out_specs=(pl.BlockSpec((HW, width), lambda n: (n, 0)),
                   pl.BlockSpec((1, 1, width), lambda n: (n, 0, 0)),
                   pl.BlockSpec((1, 1, width), lambda n: (n, 0, 0))),
        scratch_shapes=[pltpu.VMEM((HP_ROWS, C), f32),
                        pltpu.VMEM((HW, 9 * width), f32)],
        compiler_params=params,
    )(xc, a1, w1m, w2m)
    a2 = _bn_fold(jnp.sum(s2p, axis=0), jnp.sum(q2p, axis=0), g2, b2, M)

    s3p, q3p = pl.pallas_call(
        _k_stats2,
        out_shape=(sds((nm, 1, c_out), f32), sds((nm, 1, c_out), f32)),
        grid=(nm,),
        in_specs=[pl.BlockSpec((TM, width), lambda m: (m, 0)),
                  pl.BlockSpec((2, width), lambda m: (0, 0)),
                  pl.BlockSpec((width, c_out), lambda m: (0, 0))],
        out_specs=(pl.BlockSpec((1, 1, c_out), lambda m: (m, 0, 0)),
                   pl.BlockSpec((1, 1, c_out), lambda m: (m, 0, 0))),
        compiler_params=params,
    )(y2, a2, w3m)
    a3 = _bn_fold(jnp.sum(s3p, axis=0), jnp.sum(q3p, axis=0), g3, b3, M)

    out = pl.pallas_call(
        _k_out,
        out_shape=sds((N, c_out, HW), f32),
        grid=(N,),
        in_specs=[pl.BlockSpec((HW, width), lambda n: (n, 0)),
                  pl.BlockSpec((2, width), lambda n: (0, 0)),
                  pl.BlockSpec((width, c_out), lambda n: (0, 0)),
                  pl.BlockSpec((2, c_out), lambda n: (0, 0)),
                  pl.BlockSpec((1, C, HW), lambda n: (n, 0, 0))],
        out_specs=pl.BlockSpec((1, c_out, HW), lambda n: (n, 0, 0)),
        compiler_params=params,
    )(y2, a2, w3m, a3, xc)

    return out.reshape(N, c_out, H, W)


# speedup vs baseline: 4.6819x; 1.0315x over previous
import jax
import jax.numpy as jnp
from jax.experimental import pallas as pl
from jax.experimental.pallas import tpu as pltpu

_EPS = 1e-5
_VMEM_LIMIT = 48 * 1024 * 1024


def _bn_fold(s, q, gamma, beta, m_real):
    mean = s / float(m_real)
    var = jnp.maximum(q / float(m_real) - mean * mean, 0.0)
    rstd = jax.lax.rsqrt(var + _EPS)
    scale = gamma.reshape(1, -1) * rstd
    shift = beta.reshape(1, -1) - mean * scale
    return jnp.concatenate([scale, shift], axis=0)


def _k_gram(x_ref, g_ref, sx_ref):
    b = x_ref.shape[0]
    acc = jnp.zeros((x_ref.shape[1], x_ref.shape[1]), jnp.float32)
    sx = jnp.zeros((x_ref.shape[1], 1), jnp.float32)
    for i in range(b):
        xi = x_ref[i]
        acc = acc + jax.lax.dot_general(
            xi, xi, (((1,), (1,)), ((), ())),
            preferred_element_type=jnp.float32)
        sx = sx + jnp.sum(xi, axis=1, keepdims=True)
    g_ref[...] = acc[None]
    sx_ref[...] = sx[None]


def _k_stats2(y_ref, a_ref, w_ref, s_ref, q_ref):
    h = jnp.maximum(
        y_ref[...].astype(jnp.float32) * a_ref[0:1, :] + a_ref[1:2, :], 0.0)
    y3 = jnp.dot(h, w_ref[...], preferred_element_type=jnp.float32)
    s_ref[...] = jnp.sum(y3, axis=0, keepdims=True)[None]
    q_ref[...] = jnp.sum(y3 * y3, axis=0, keepdims=True)[None]


def _make_conv2_kernel(H, W, C, HP_ROWS, BASE):
    HW = H * W

    def _k_conv2(x_ref, a_ref, w1_ref, w2_ref, y2_ref, s_ref, q_ref,
                 hp_ref, pat_ref):
        y1 = jax.lax.dot_general(
            x_ref[0], w1_ref[...], (((0,), (0,)), ((), ())),
            preferred_element_type=jnp.float32)
        h = jnp.maximum(y1 * a_ref[0:1, :] + a_ref[1:2, :], 0.0)
        hp_ref[0:BASE, :] = jnp.zeros((BASE, h.shape[1]), jnp.float32)
        hp_ref[BASE + HW:HP_ROWS, :] = jnp.zeros(
            (HP_ROWS - BASE - HW, h.shape[1]), jnp.float32)
        hp_ref[BASE:BASE + HW, :] = h
        col = jax.lax.broadcasted_iota(jnp.int32, (HW, 1), 0) % W
        m0 = (col != 0).astype(jnp.float32)
        m2 = (col != W - 1).astype(jnp.float32)
        CW = h.shape[1]
        for di in range(3):
            for dj in range(3):
                t = 3 * di + dj
                off = W * di + dj
                win = hp_ref[off:off + HW, :]
                if dj == 0:
                    win = win * m0
                elif dj == 2:
                    win = win * m2
                pat_ref[:, t * CW:(t + 1) * CW] = win
        y2 = jnp.dot(pat_ref[...], w2_ref[...],
                     preferred_element_type=jnp.float32)
        y2_ref[...] = y2.astype(y2_ref.dtype)
        s_ref[...] = jnp.sum(y2, axis=0, keepdims=True)[None]
        q_ref[...] = jnp.sum(y2 * y2, axis=0, keepdims=True)[None]

    return _k_conv2


def _k_out(y2_ref, a2_ref, w3_ref, a3_ref, x_ref, o_ref):
    h = jnp.maximum(
        y2_ref[...].astype(jnp.float32) * a2_ref[0:1, :] + a2_ref[1:2, :], 0.0)
    y3 = jnp.dot(h, w3_ref[...], preferred_element_type=jnp.float32)
    res = jnp.transpose(x_ref[0], (1, 0))
    o = jnp.maximum(y3 * a3_ref[0:1, :] + a3_ref[1:2, :] + res, 0.0)
    o_ref[...] = jnp.transpose(o, (1, 0))[None]


def kernel(x_nchw, w1, g1, b1, w2, g2, b2, w3, b3c, g3, b3,
           wd, bd, gd, betad):
    N, C, H, W = x_nchw.shape
    HW = H * W
    M = N * HW
    f32 = jnp.float32

    xc = x_nchw.reshape(N, C, HW).astype(f32)
    width = w1.shape[0]
    c_out = w3.shape[0]
    w1m = w1.reshape(width, C).T.astype(f32)
    w2m = jnp.transpose(w2.astype(f32), (2, 3, 1, 0)).reshape(9 * width, width)
    w3m = w3.reshape(c_out, width).T.astype(f32)

    for TM in (512, 448, 256):
        if M % TM == 0:
            break
    else:
        TM = HW
    nm = M // TM

    GB = 4
    while N % GB:
        GB //= 2
    ng = N // GB

    BASE = W + 1
    HP_ROWS = -(-(2 * W + 2 + HW + BASE) // 8) * 8

    sds = jax.ShapeDtypeStruct
    params = pltpu.CompilerParams(
        dimension_semantics=("parallel",), vmem_limit_bytes=_VMEM_LIMIT)

    gp, sxp = pl.pallas_call(
        _k_gram,
        out_shape=(sds((ng, C, C), f32), sds((ng, C, 1), f32)),
        grid=(ng,),
        in_specs=[pl.BlockSpec((GB, C, HW), lambda i: (i, 0, 0))],
        out_specs=(pl.BlockSpec((1, C, C), lambda i: (i, 0, 0)),
                   pl.BlockSpec((1, C, 1), lambda i: (i, 0, 0))),
        compiler_params=params,
    )(xc)
    g1m = jnp.sum(gp, axis=0)
    sx = jnp.sum(sxp, axis=0).reshape(1, C)
    s1 = sx @ w1m
    q1 = jnp.sum(w1m * (g1m @ w1m), axis=0).reshape(1, width)
    a1 = _bn_fold(s1, q1, g1, b1, M)

    y2, s2p, q2p = pl.pallas_call(
        _make_conv2_kernel(H, W, C, HP_ROWS, BASE),
        out_shape=(sds((M, width), jnp.bfloat16),
                   sds((N, 1, width), f32),
                   sds((N, 1, width), f32)),
        grid=(N,),
        in_specs=[pl.BlockSpec((1, C, HW), lambda n: (n, 0, 0)),
                  pl.BlockSpec((2, width), lambda n: (0, 0)),
                  pl.BlockSpec((C, width), lambda n: (0, 0)),
                  pl.BlockSpec((9 * width, width), lambda n: (0, 0))],
        out_specs=(pl.BlockSpec((HW, width), lambda n: (n, 0)),
                   pl.BlockSpec((1, 1, width), lambda n: (n, 0, 0)),
                   pl.BlockSpec((1, 1, width), lambda n: (n, 0, 0))),
        scratch_shapes=[pltpu.VMEM((HP_ROWS, C), f32),
                        pltpu.VMEM((HW, 9 * width), f32)],
        compiler_params=params,
    )(xc, a1, w1m, w2m)
    a2 = _bn_fold(jnp.sum(s2p, axis=0), jnp.sum(q2p, axis=0), g2, b2, M)

    s3p, q3p = pl.pallas_call(
        _k_stats2,
        out_shape=(sds((nm, 1, c_out), f32), sds((nm, 1, c_out), f32)),
        grid=(nm,),
        in_specs=[pl.BlockSpec((TM, width), lambda m: (m, 0)),
                  pl.BlockSpec((2, width), lambda m: (0, 0)),
                  pl.BlockSpec((width, c_out), lambda m: (0, 0))],
        out_specs=(pl.BlockSpec((1, 1, c_out), lambda m: (m, 0, 0)),
                   pl.BlockSpec((1, 1, c_out), lambda m: (m, 0, 0))),
        compiler_params=params,
    )(y2, a2, w3m)
    a3 = _bn_fold(jnp.sum(s3p, axis=0), jnp.sum(q3p, axis=0), g3, b3, M)

    out = pl.pallas_call(
        _k_out,
        out_shape=sds((N, c_out, HW), f32),
        grid=(N,),
        in_specs=[pl.BlockSpec((HW, width), lambda n: (n, 0)),
                  pl.BlockSpec((2, width), lambda n: (0, 0)),
                  pl.BlockSpec((width, c_out), lambda n: (0, 0)),
                  pl.BlockSpec((2, c_out), lambda n: (0, 0)),
                  pl.BlockSpec((1, C, HW), lambda n: (n, 0, 0))],
        out_specs=pl.BlockSpec((1, c_out, HW), lambda n: (n, 0, 0)),
        compiler_params=params,
    )(y2, a2, w3m, a3, xc)

    return out.reshape(N, c_out, H, W)


# speedup vs baseline: 6.9507x; 1.4846x over previous
import jax
import jax.numpy as jnp
from jax.experimental import pallas as pl
from jax.experimental.pallas import tpu as pltpu

_EPS = 1e-5
_VMEM_LIMIT = 48 * 1024 * 1024


def _bn_fold(s, q, gamma, beta, m_real):
    mean = s / float(m_real)
    var = jnp.maximum(q / float(m_real) - mean * mean, 0.0)
    rstd = jax.lax.rsqrt(var + _EPS)
    scale = gamma.reshape(1, -1) * rstd
    shift = beta.reshape(1, -1) - mean * scale
    return jnp.concatenate([scale, shift], axis=0)


def _k_gram(x_ref, g_ref, sx_ref):
    b = x_ref.shape[0]
    acc = jnp.zeros((x_ref.shape[1], x_ref.shape[1]), jnp.float32)
    sx = jnp.zeros((x_ref.shape[1], 1), jnp.float32)
    for i in range(b):
        xi = x_ref[i]
        acc = acc + jax.lax.dot_general(
            xi, xi, (((1,), (1,)), ((), ())),
            preferred_element_type=jnp.float32)
        sx = sx + jnp.sum(xi, axis=1, keepdims=True)
    g_ref[...] = acc[None]
    sx_ref[...] = sx[None]


def _k_stats2(y_ref, a_ref, w_ref, s_ref, q_ref):
    h = jnp.maximum(
        y_ref[...].astype(jnp.float32) * a_ref[0:1, :] + a_ref[1:2, :], 0.0)
    y3 = jnp.dot(h, w_ref[...], preferred_element_type=jnp.float32)
    s_ref[...] = jnp.sum(y3, axis=0, keepdims=True)[None]
    q_ref[...] = jnp.sum(y3 * y3, axis=0, keepdims=True)[None]


def _make_conv2_kernel(H, W, C, HP_ROWS, BASE, IMG_T):
    HW = H * W

    def _k_conv2(x_ref, a_ref, w1_ref, w2_ref, y2_ref, s_ref, q_ref,
                 hp_ref, pat_ref):
        col = jax.lax.broadcasted_iota(jnp.int32, (HW, 1), 0) % W
        m0 = (col != 0).astype(jnp.float32)
        m2 = (col != W - 1).astype(jnp.float32)
        for i in range(IMG_T):
            y1 = jax.lax.dot_general(
                x_ref[i], w1_ref[...], (((0,), (0,)), ((), ())),
                preferred_element_type=jnp.float32)
            h = jnp.maximum(y1 * a_ref[0:1, :] + a_ref[1:2, :], 0.0)
            CW = h.shape[1]
            hp_ref[0:BASE, :] = jnp.zeros((BASE, CW), jnp.float32)
            hp_ref[BASE + HW:HP_ROWS, :] = jnp.zeros(
                (HP_ROWS - BASE - HW, CW), jnp.float32)
            hp_ref[BASE:BASE + HW, :] = h
            for di in range(3):
                for dj in range(3):
                    t = 3 * di + dj
                    off = W * di + dj
                    win = hp_ref[off:off + HW, :]
                    if dj == 0:
                        win = win * m0
                    elif dj == 2:
                        win = win * m2
                    pat_ref[i * HW:(i + 1) * HW, t * CW:(t + 1) * CW] = win
        y2 = jnp.dot(pat_ref[...], w2_ref[...],
                     preferred_element_type=jnp.float32)
        y2_ref[...] = y2.astype(y2_ref.dtype)
        s_ref[...] = jnp.sum(y2, axis=0, keepdims=True)[None]
        q_ref[...] = jnp.sum(y2 * y2, axis=0, keepdims=True)[None]

    return _k_conv2


def _make_out_kernel(HW, IMG_T):
    def _k_out(y2_ref, a2_ref, w3_ref, a3_ref, x_ref, o_ref):
        h = jnp.maximum(
            y2_ref[...].astype(jnp.float32) * a2_ref[0:1, :] + a2_ref[1:2, :],
            0.0)
        y3 = jnp.dot(h, w3_ref[...], preferred_element_type=jnp.float32)
        z = y3 * a3_ref[0:1, :] + a3_ref[1:2, :]
        for i in range(IMG_T):
            res = jnp.transpose(x_ref[i], (1, 0))
            o = jnp.maximum(z[i * HW:(i + 1) * HW, :] + res, 0.0)
            o_ref[i] = jnp.transpose(o, (1, 0))

    return _k_out


def kernel(x_nchw, w1, g1, b1, w2, g2, b2, w3, b3c, g3, b3,
           wd, bd, gd, betad):
    N, C, H, W = x_nchw.shape
    HW = H * W
    M = N * HW
    f32 = jnp.float32

    xc = x_nchw.reshape(N, C, HW).astype(f32)
    width = w1.shape[0]
    c_out = w3.shape[0]
    w1m = w1.reshape(width, C).T.astype(f32)
    w2m = jnp.transpose(w2.astype(f32), (2, 3, 1, 0)).reshape(9 * width, width)
    w3m = w3.reshape(c_out, width).T.astype(f32)

    IMG_T = 4
    while N % IMG_T:
        IMG_T //= 2
    nt = N // IMG_T
    TB = IMG_T * HW

    TM = TB
    nm = M // TM

    BASE = W + 1
    HP_ROWS = -(-(2 * W + 2 + HW + BASE) // 8) * 8

    sds = jax.ShapeDtypeStruct
    params = pltpu.CompilerParams(
        dimension_semantics=("parallel",), vmem_limit_bytes=_VMEM_LIMIT)

    gp, sxp = pl.pallas_call(
        _k_gram,
        out_shape=(sds((nt, C, C), f32), sds((nt, C, 1), f32)),
        grid=(nt,),
        in_specs=[pl.BlockSpec((IMG_T, C, HW), lambda i: (i, 0, 0))],
        out_specs=(pl.BlockSpec((1, C, C), lambda i: (i, 0, 0)),
                   pl.BlockSpec((1, C, 1), lambda i: (i, 0, 0))),
        compiler_params=params,
    )(xc)
    g1m = jnp.sum(gp, axis=0)
    sx = jnp.sum(sxp, axis=0).reshape(1, C)
    s1 = sx @ w1m
    q1 = jnp.sum(w1m * (g1m @ w1m), axis=0).reshape(1, width)
    a1 = _bn_fold(s1, q1, g1, b1, M)

    y2, s2p, q2p = pl.pallas_call(
        _make_conv2_kernel(H, W, C, HP_ROWS, BASE, IMG_T),
        out_shape=(sds((M, width), jnp.bfloat16),
                   sds((nt, 1, width), f32),
                   sds((nt, 1, width), f32)),
        grid=(nt,),
        in_specs=[pl.BlockSpec((IMG_T, C, HW), lambda n: (n, 0, 0)),
                  pl.BlockSpec((2, width), lambda n: (0, 0)),
                  pl.BlockSpec((C, width), lambda n: (0, 0)),
                  pl.BlockSpec((9 * width, width), lambda n: (0, 0))],
        out_specs=(pl.BlockSpec((TB, width), lambda n: (n, 0)),
                   pl.BlockSpec((1, 1, width), lambda n: (n, 0, 0)),
                   pl.BlockSpec((1, 1, width), lambda n: (n, 0, 0))),
        scratch_shapes=[pltpu.VMEM((HP_ROWS, C), f32),
                        pltpu.VMEM((TB, 9 * width), f32)],
        compiler_params=params,
    )(xc, a1, w1m, w2m)
    a2 = _bn_fold(jnp.sum(s2p, axis=0), jnp.sum(q2p, axis=0), g2, b2, M)

    s3p, q3p = pl.pallas_call(
        _k_stats2,
        out_shape=(sds((nm, 1, c_out), f32), sds((nm, 1, c_out), f32)),
        grid=(nm,),
        in_specs=[pl.BlockSpec((TM, width), lambda m: (m, 0)),
                  pl.BlockSpec((2, width), lambda m: (0, 0)),
                  pl.BlockSpec((width, c_out), lambda m: (0, 0))],
        out_specs=(pl.BlockSpec((1, 1, c_out), lambda m: (m, 0, 0)),
                   pl.BlockSpec((1, 1, c_out), lambda m: (m, 0, 0))),
        compiler_params=params,
    )(y2, a2, w3m)
    a3 = _bn_fold(jnp.sum(s3p, axis=0), jnp.sum(q3p, axis=0), g3, b3, M)

    out = pl.pallas_call(
        _make_out_kernel(HW, IMG_T),
        out_shape=sds((N, c_out, HW), f32),
        grid=(nt,),
        in_specs=[pl.BlockSpec((TB, width), lambda n: (n, 0)),
                  pl.BlockSpec((2, width), lambda n: (0, 0)),
                  pl.BlockSpec((width, c_out), lambda n: (0, 0)),
                  pl.BlockSpec((2, c_out), lambda n: (0, 0)),
                  pl.BlockSpec((IMG_T, C, HW), lambda n: (n, 0, 0))],
        out_specs=pl.BlockSpec((IMG_T, c_out, HW), lambda n: (n, 0, 0)),
        compiler_params=params,
    )(y2, a2, w3m, a3, xc)

    return out.reshape(N, c_out, H, W)


# speedup vs baseline: 7.4532x; 1.0723x over previous
import jax
import jax.numpy as jnp
from jax.experimental import pallas as pl
from jax.experimental.pallas import tpu as pltpu

_EPS = 1e-5
_VMEM_LIMIT = 48 * 1024 * 1024


def _bn_fold(s, q, gamma, beta, m_real):
    mean = s / float(m_real)
    var = jnp.maximum(q / float(m_real) - mean * mean, 0.0)
    rstd = jax.lax.rsqrt(var + _EPS)
    scale = gamma.reshape(1, -1) * rstd
    shift = beta.reshape(1, -1) - mean * scale
    return jnp.concatenate([scale, shift], axis=0)


def _k_gram(x_ref, g_ref, sx_ref):
    b = x_ref.shape[0]
    acc = jnp.zeros((x_ref.shape[1], x_ref.shape[1]), jnp.float32)
    sx = jnp.zeros((x_ref.shape[1], 1), jnp.float32)
    for i in range(b):
        xi = x_ref[i]
        acc = acc + jax.lax.dot_general(
            xi, xi, (((1,), (1,)), ((), ())),
            preferred_element_type=jnp.float32)
        sx = sx + jnp.sum(xi, axis=1, keepdims=True)
    g_ref[...] = acc[None]
    sx_ref[...] = sx[None]


def _k_stats2(y_ref, a_ref, w_ref, s_ref, q_ref):
    h = jnp.maximum(
        y_ref[...].astype(jnp.float32) * a_ref[0:1, :] + a_ref[1:2, :], 0.0)
    y3 = jnp.dot(h, w_ref[...], preferred_element_type=jnp.float32)
    s_ref[...] = jnp.sum(y3, axis=0, keepdims=True)[None]
    q_ref[...] = jnp.sum(y3 * y3, axis=0, keepdims=True)[None]


def _make_conv2_kernel(H, W, C, HP_ROWS, BASE, IMG_T):
    HW = H * W

    def _k_conv2(x_ref, a_ref, w1_ref, w2_ref, y2_ref, s_ref, q_ref,
                 hp_ref, pat_ref):
        col = jax.lax.broadcasted_iota(jnp.int32, (HW, 1), 0) % W
        m0 = (col != 0).astype(jnp.float32)
        m2 = (col != W - 1).astype(jnp.float32)
        for i in range(IMG_T):
            y1 = jax.lax.dot_general(
                x_ref[i], w1_ref[...], (((0,), (0,)), ((), ())),
                preferred_element_type=jnp.float32)
            h = jnp.maximum(y1 * a_ref[0:1, :] + a_ref[1:2, :], 0.0)
            CW = h.shape[1]
            hp_ref[0:BASE, :] = jnp.zeros((BASE, CW), jnp.float32)
            hp_ref[BASE + HW:HP_ROWS, :] = jnp.zeros(
                (HP_ROWS - BASE - HW, CW), jnp.float32)
            hp_ref[BASE:BASE + HW, :] = h
            for di in range(3):
                for dj in range(3):
                    t = 3 * di + dj
                    off = W * di + dj
                    win = hp_ref[off:off + HW, :]
                    if dj == 0:
                        win = win * m0
                    elif dj == 2:
                        win = win * m2
                    pat_ref[i * HW:(i + 1) * HW, t * CW:(t + 1) * CW] = win
        y2 = jnp.dot(pat_ref[...], w2_ref[...],
                     preferred_element_type=jnp.float32)
        y2_ref[...] = y2.astype(y2_ref.dtype)
        s_ref[...] = jnp.sum(y2, axis=0, keepdims=True)[None]
        q_ref[...] = jnp.sum(y2 * y2, axis=0, keepdims=True)[None]

    return _k_conv2


def _make_out_kernel(HW, IMG_T):
    def _k_out(y2_ref, a2_ref, w3_ref, a3_ref, x_ref, o_ref):
        h = jnp.maximum(
            y2_ref[...].astype(jnp.float32) * a2_ref[0:1, :] + a2_ref[1:2, :],
            0.0)
        y3 = jnp.dot(h, w3_ref[...], preferred_element_type=jnp.float32)
        z = y3 * a3_ref[0:1, :] + a3_ref[1:2, :]
        for i in range(IMG_T):
            res = jnp.transpose(x_ref[i], (1, 0))
            o = jnp.maximum(z[i * HW:(i + 1) * HW, :] + res, 0.0)
            o_ref[i] = jnp.transpose(o, (1, 0))

    return _k_out


def kernel(x_nchw, w1, g1, b1, w2, g2, b2, w3, b3c, g3, b3,
           wd, bd, gd, betad):
    N, C, H, W = x_nchw.shape
    HW = H * W
    M = N * HW
    f32 = jnp.float32

    xc = x_nchw.reshape(N, C, HW).astype(f32)
    width = w1.shape[0]
    c_out = w3.shape[0]
    w1m = w1.reshape(width, C).T.astype(f32)
    w2m = jnp.transpose(w2.astype(f32), (2, 3, 1, 0)).reshape(9 * width, width)
    w3m = w3.reshape(c_out, width).T.astype(f32)

    IMG_T = 8
    while N % IMG_T:
        IMG_T //= 2
    nt = N // IMG_T
    TB = IMG_T * HW

    TM = TB
    nm = M // TM

    BASE = W + 1
    HP_ROWS = -(-(2 * W + 2 + HW + BASE) // 8) * 8

    sds = jax.ShapeDtypeStruct
    params = pltpu.CompilerParams(
        dimension_semantics=("parallel",), vmem_limit_bytes=_VMEM_LIMIT)

    gp, sxp = pl.pallas_call(
        _k_gram,
        out_shape=(sds((nt, C, C), f32), sds((nt, C, 1), f32)),
        grid=(nt,),
        in_specs=[pl.BlockSpec((IMG_T, C, HW), lambda i: (i, 0, 0))],
        out_specs=(pl.BlockSpec((1, C, C), lambda i: (i, 0, 0)),
                   pl.BlockSpec((1, C, 1), lambda i: (i, 0, 0))),
        compiler_params=params,
    )(xc)
    g1m = jnp.sum(gp, axis=0)
    sx = jnp.sum(sxp, axis=0).reshape(1, C)
    s1 = sx @ w1m
    q1 = jnp.sum(w1m * (g1m @ w1m), axis=0).reshape(1, width)
    a1 = _bn_fold(s1, q1, g1, b1, M)

    y2, s2p, q2p = pl.pallas_call(
        _make_conv2_kernel(H, W, C, HP_ROWS, BASE, IMG_T),
        out_shape=(sds((M, width), jnp.bfloat16),
                   sds((nt, 1, width), f32),
                   sds((nt, 1, width), f32)),
        grid=(nt,),
        in_specs=[pl.BlockSpec((IMG_T, C, HW), lambda n: (n, 0, 0)),
                  pl.BlockSpec((2, width), lambda n: (0, 0)),
                  pl.BlockSpec((C, width), lambda n: (0, 0)),
                  pl.BlockSpec((9 * width, width), lambda n: (0, 0))],
        out_specs=(pl.BlockSpec((TB, width), lambda n: (n, 0)),
                   pl.BlockSpec((1, 1, width), lambda n: (n, 0, 0)),
                   pl.BlockSpec((1, 1, width), lambda n: (n, 0, 0))),
        scratch_shapes=[pltpu.VMEM((HP_ROWS, C), f32),
                        pltpu.VMEM((TB, 9 * width), f32)],
        compiler_params=params,
    )(xc, a1, w1m, w2m)
    a2 = _bn_fold(jnp.sum(s2p, axis=0), jnp.sum(q2p, axis=0), g2, b2, M)

    s3p, q3p = pl.pallas_call(
        _k_stats2,
        out_shape=(sds((nm, 1, c_out), f32), sds((nm, 1, c_out), f32)),
        grid=(nm,),
        in_specs=[pl.BlockSpec((TM, width), lambda m: (m, 0)),
                  pl.BlockSpec((2, width), lambda m: (0, 0)),
                  pl.BlockSpec((width, c_out), lambda m: (0, 0))],
        out_specs=(pl.BlockSpec((1, 1, c_out), lambda m: (m, 0, 0)),
                   pl.BlockSpec((1, 1, c_out), lambda m: (m, 0, 0))),
        compiler_params=params,
    )(y2, a2, w3m)
    a3 = _bn_fold(jnp.sum(s3p, axis=0), jnp.sum(q3p, axis=0), g3, b3, M)

    out = pl.pallas_call(
        _make_out_kernel(HW, IMG_T),
        out_shape=sds((N, c_out, HW), f32),
        grid=(nt,),
        in_specs=[pl.BlockSpec((TB, width), lambda n: (n, 0)),
                  pl.BlockSpec((2, width), lambda n: (0, 0)),
                  pl.BlockSpec((width, c_out), lambda n: (0, 0)),
                  pl.BlockSpec((2, c_out), lambda n: (0, 0)),
                  pl.BlockSpec((IMG_T, C, HW), lambda n: (n, 0, 0))],
        out_specs=pl.BlockSpec((IMG_T, c_out, HW), lambda n: (n, 0, 0)),
        compiler_params=params,
    )(y2, a2, w3m, a3, xc)

    return out.reshape(N, c_out, H, W)


# speedup vs baseline: 7.8250x; 1.0499x over previous
import jax
import jax.numpy as jnp
from jax.experimental import pallas as pl
from jax.experimental.pallas import tpu as pltpu

_EPS = 1e-5
_VMEM_LIMIT = 48 * 1024 * 1024


def _bn_fold(s, q, gamma, beta, m_real):
    mean = s / float(m_real)
    var = jnp.maximum(q / float(m_real) - mean * mean, 0.0)
    rstd = jax.lax.rsqrt(var + _EPS)
    scale = gamma.reshape(1, -1) * rstd
    shift = beta.reshape(1, -1) - mean * scale
    return jnp.concatenate([scale, shift], axis=0)


def _make_p0_kernel(HW, IMG_T):
    def _k_p0(x_ref, w1_ref, y1_ref, s_ref, q_ref):
        width = w1_ref.shape[1]
        s = jnp.zeros((1, width), jnp.float32)
        q = jnp.zeros((1, width), jnp.float32)
        for i in range(IMG_T):
            y1 = jax.lax.dot_general(
                x_ref[i], w1_ref[...], (((0,), (0,)), ((), ())),
                preferred_element_type=jnp.float32)
            y1_ref[i * HW:(i + 1) * HW, :] = y1.astype(y1_ref.dtype)
            s = s + jnp.sum(y1, axis=0, keepdims=True)
            q = q + jnp.sum(y1 * y1, axis=0, keepdims=True)
        s_ref[...] = s[None]
        q_ref[...] = q[None]

    return _k_p0


def _k_stats2(y_ref, a_ref, w_ref, s_ref, q_ref):
    h = jnp.maximum(
        y_ref[...].astype(jnp.float32) * a_ref[0:1, :] + a_ref[1:2, :], 0.0)
    y3 = jnp.dot(h, w_ref[...], preferred_element_type=jnp.float32)
    s_ref[...] = jnp.sum(y3, axis=0, keepdims=True)[None]
    q_ref[...] = jnp.sum(y3 * y3, axis=0, keepdims=True)[None]


def _make_conv2_kernel(H, W, HP_ROWS, BASE, IMG_T):
    HW = H * W

    def _k_conv2(y1_ref, a_ref, w2_ref, y2_ref, s_ref, q_ref,
                 hp_ref, pat_ref):
        CW = w2_ref.shape[1]
        col = jax.lax.broadcasted_iota(jnp.int32, (HW, 1), 0) % W
        m0 = (col != 0).astype(jnp.float32)
        m2 = (col != W - 1).astype(jnp.float32)
        for i in range(IMG_T):
            h = jnp.maximum(
                y1_ref[i * HW:(i + 1) * HW, :].astype(jnp.float32)
                * a_ref[0:1, :] + a_ref[1:2, :], 0.0)
            hp_ref[0:BASE, :] = jnp.zeros((BASE, CW), jnp.float32)
            hp_ref[BASE + HW:HP_ROWS, :] = jnp.zeros(
                (HP_ROWS - BASE - HW, CW), jnp.float32)
            hp_ref[BASE:BASE + HW, :] = h
            for di in range(3):
                for dj in range(3):
                    t = 3 * di + dj
                    off = W * di + dj
                    win = hp_ref[off:off + HW, :]
                    if dj == 0:
                        win = win * m0
                    elif dj == 2:
                        win = win * m2
                    pat_ref[i * HW:(i + 1) * HW, t * CW:(t + 1) * CW] = win
        y2 = jnp.dot(pat_ref[...], w2_ref[...],
                     preferred_element_type=jnp.float32)
        y2_ref[...] = y2.astype(y2_ref.dtype)
        s_ref[...] = jnp.sum(y2, axis=0, keepdims=True)[None]
        q_ref[...] = jnp.sum(y2 * y2, axis=0, keepdims=True)[None]

    return _k_conv2


def _make_out_kernel(HW, IMG_T):
    def _k_out(y2_ref, a2_ref, w3_ref, a3_ref, x_ref, o_ref):
        h = jnp.maximum(
            y2_ref[...].astype(jnp.float32) * a2_ref[0:1, :] + a2_ref[1:2, :],
            0.0)
        y3 = jnp.dot(h, w3_ref[...], preferred_element_type=jnp.float32)
        z = y3 * a3_ref[0:1, :] + a3_ref[1:2, :]
        for i in range(IMG_T):
            res = jnp.transpose(x_ref[i], (1, 0))
            o = jnp.maximum(z[i * HW:(i + 1) * HW, :] + res, 0.0)
            o_ref[i] = jnp.transpose(o, (1, 0))

    return _k_out


def kernel(x_nchw, w1, g1, b1, w2, g2, b2, w3, b3c, g3, b3,
           wd, bd, gd, betad):
    N, C, H, W = x_nchw.shape
    HW = H * W
    M = N * HW
    f32 = jnp.float32

    xc = x_nchw.reshape(N, C, HW).astype(f32)
    width = w1.shape[0]
    c_out = w3.shape[0]
    w1m = w1.reshape(width, C).T.astype(f32)
    w2m = jnp.transpose(w2.astype(f32), (2, 3, 1, 0)).reshape(9 * width, width)
    w3m = w3.reshape(c_out, width).T.astype(f32)

    IMG_T = 8
    while N % IMG_T:
        IMG_T //= 2
    nt = N // IMG_T
    TB = IMG_T * HW

    TM = TB
    nm = M // TM

    BASE = W + 1
    HP_ROWS = -(-(2 * W + 2 + HW + BASE) // 8) * 8

    sds = jax.ShapeDtypeStruct
    params = pltpu.CompilerParams(
        dimension_semantics=("parallel",), vmem_limit_bytes=_VMEM_LIMIT)

    y1, s1p, q1p = pl.pallas_call(
        _make_p0_kernel(HW, IMG_T),
        out_shape=(sds((M, width), jnp.bfloat16),
                   sds((nt, 1, width), f32),
                   sds((nt, 1, width), f32)),
        grid=(nt,),
        in_specs=[pl.BlockSpec((IMG_T, C, HW), lambda i: (i, 0, 0)),
                  pl.BlockSpec((C, width), lambda i: (0, 0))],
        out_specs=(pl.BlockSpec((TB, width), lambda i: (i, 0)),
                   pl.BlockSpec((1, 1, width), lambda i: (i, 0, 0)),
                   pl.BlockSpec((1, 1, width), lambda i: (i, 0, 0))),
        compiler_params=params,
    )(xc, w1m)
    a1 = _bn_fold(jnp.sum(s1p, axis=0), jnp.sum(q1p, axis=0), g1, b1, M)

    y2, s2p, q2p = pl.pallas_call(
        _make_conv2_kernel(H, W, HP_ROWS, BASE, IMG_T),
        out_shape=(sds((M, width), jnp.bfloat16),
                   sds((nt, 1, width), f32),
                   sds((nt, 1, width), f32)),
        grid=(nt,),
        in_specs=[pl.BlockSpec((TB, width), lambda n: (n, 0)),
                  pl.BlockSpec((2, width), lambda n: (0, 0)),
                  pl.BlockSpec((9 * width, width), lambda n: (0, 0))],
        out_specs=(pl.BlockSpec((TB, width), lambda n: (n, 0)),
                   pl.BlockSpec((1, 1, width), lambda n: (n, 0, 0)),
                   pl.BlockSpec((1, 1, width), lambda n: (n, 0, 0))),
        scratch_shapes=[pltpu.VMEM((HP_ROWS, width), f32),
                        pltpu.VMEM((TB, 9 * width), f32)],
        compiler_params=params,
    )(y1, a1, w2m)
    a2 = _bn_fold(jnp.sum(s2p, axis=0), jnp.sum(q2p, axis=0), g2, b2, M)

    s3p, q3p = pl.pallas_call(
        _k_stats2,
        out_shape=(sds((nm, 1, c_out), f32), sds((nm, 1, c_out), f32)),
        grid=(nm,),
        in_specs=[pl.BlockSpec((TM, width), lambda m: (m, 0)),
                  pl.BlockSpec((2, width), lambda m: (0, 0)),
                  pl.BlockSpec((width, c_out), lambda m: (0, 0))],
        out_specs=(pl.BlockSpec((1, 1, c_out), lambda m: (m, 0, 0)),
                   pl.BlockSpec((1, 1, c_out), lambda m: (m, 0, 0))),
        compiler_params=params,
    )(y2, a2, w3m)
    a3 = _bn_fold(jnp.sum(s3p, axis=0), jnp.sum(q3p, axis=0), g3, b3, M)

    out = pl.pallas_call(
        _make_out_kernel(HW, IMG_T),
        out_shape=sds((N, c_out, HW), f32),
        grid=(nt,),
        in_specs=[pl.BlockSpec((TB, width), lambda n: (n, 0)),
                  pl.BlockSpec((2, width), lambda n: (0, 0)),
                  pl.BlockSpec((width, c_out), lambda n: (0, 0)),
                  pl.BlockSpec((2, c_out), lambda n: (0, 0)),
                  pl.BlockSpec((IMG_T, C, HW), lambda n: (n, 0, 0))],
        out_specs=pl.BlockSpec((IMG_T, c_out, HW), lambda n: (n, 0, 0)),
        compiler_params=params,
    )(y2, a2, w3m, a3, xc)

    return out.reshape(N, c_out, H, W)
